# baseline (speedup 1.0000x reference)
import functools
import math

import jax
import jax.numpy as jnp
from jax import lax
from jax.experimental import pallas as pl
from jax.experimental.pallas import tpu as pltpu
from jax.experimental.pallas import tpu_sc as plsc

EMBED_DIM = 768
FFN_DIM = 3072
NUM_EXPERTS = 8
TOP_K = 2
LB_W = 0.01
Z_W = 0.001

SEQ = 2048
BLK = 256
CAP = TOP_K * SEQ + NUM_EXPERTS * BLK
NB = CAP // BLK
NPAIRS = TOP_K * SEQ
NWORKERS = 32
PW = NPAIRS // NWORKERS
TW = SEQ // NWORKERS

_INV_SQRT2 = 1.0 / math.sqrt(2.0)


def _gelu_exact(h):
    return 0.5 * h * (1.0 + jax.lax.erf(h * _INV_SQRT2))


def _cumsum_rows(a):
    n = a.shape[0]
    s = 1
    while s < n:
        shifted = jnp.concatenate(
            [jnp.zeros((s,) + a.shape[1:], a.dtype), a[:n - s]], axis=0)
        a = a + shifted
        s *= 2
    return a


def _router_body(x_ref, wr_ref, dest_ref, wk_ref, segblk_ref, nblk_ref,
                 lb_ref, z_ref):
    xv = x_ref[...]
    logits = jnp.dot(xv, wr_ref[...], preferred_element_type=jnp.float32)
    S, E = logits.shape
    m = jnp.max(logits, axis=1, keepdims=True)
    ex = jnp.exp(logits - m)
    sm = jnp.sum(ex, axis=1, keepdims=True)
    probs = ex / sm
    iota8 = jax.lax.broadcasted_iota(jnp.int32, (S, E), 1)
    m1 = jnp.max(probs, axis=1, keepdims=True)
    i1 = jnp.min(jnp.where(probs == m1, iota8, E), axis=1, keepdims=True)
    probs_m = jnp.where(iota8 == i1, -1.0, probs)
    m2 = jnp.max(probs_m, axis=1, keepdims=True)
    i2 = jnp.min(jnp.where(probs_m == m2, iota8, E), axis=1, keepdims=True)
    denom = m1 + m2 + 1e-8
    wk_ref[...] = jnp.concatenate([m1 / denom, m2 / denom], axis=1)

    usage = jnp.mean(probs, axis=0, keepdims=True)
    mean_u = jnp.mean(usage)
    var_u = jnp.mean(jnp.square(usage - mean_u))
    lb_ref[...] = jnp.reshape(
        var_u / (mean_u * mean_u + 1e-8) * (NUM_EXPERTS * LB_W), (1, 1))
    lse = m[:, 0] + jnp.log(jnp.sum(ex, axis=1))
    z_ref[...] = jnp.reshape(jnp.mean(jnp.square(lse)) * Z_W, (1, 1))

    oh1 = (iota8 == i1).astype(jnp.float32)
    oh2 = (iota8 == i2).astype(jnp.float32)
    c1 = _cumsum_rows(oh1)
    c2 = _cumsum_rows(oh2)
    cnt1 = c1[S - 1:S, :]
    cnt2 = c2[S - 1:S, :]
    rank1 = jnp.sum(oh1 * c1, axis=1, keepdims=True) - 1.0
    rank2 = jnp.sum(oh2 * c2, axis=1, keepdims=True) - 1.0
    cnt = (cnt1 + cnt2).astype(jnp.int32)
    capi = ((cnt + (BLK - 1)) // BLK) * BLK
    csum = capi
    for s in (1, 2, 4):
        csum = csum + jnp.concatenate(
            [jnp.zeros((1, s), jnp.int32), csum[:, :NUM_EXPERTS - s]], axis=1)
    seg_start = csum - capi
    segf = seg_start.astype(jnp.float32)
    base1 = jnp.sum(oh1 * segf, axis=1, keepdims=True)
    base2 = jnp.sum(oh2 * (segf + cnt1), axis=1, keepdims=True)
    dest1 = (base1 + rank1).astype(jnp.int32)
    dest2 = (base2 + rank2).astype(jnp.int32)
    dest_ref[...] = jnp.concatenate([dest1, dest2], axis=1)

    segblk_ref[...] = seg_start // BLK
    nblk_ref[...] = capi // BLK


@jax.jit
def _router_call(x2d, Wr):
    return pl.pallas_call(
        _router_body,
        in_specs=[
            pl.BlockSpec((SEQ, EMBED_DIM), lambda: (0, 0)),
            pl.BlockSpec((EMBED_DIM, NUM_EXPERTS), lambda: (0, 0)),
        ],
        out_specs=[
            pl.BlockSpec((SEQ, TOP_K), lambda: (0, 0)),
            pl.BlockSpec((SEQ, TOP_K), lambda: (0, 0)),
            pl.BlockSpec((1, NUM_EXPERTS), lambda: (0, 0)),
            pl.BlockSpec((1, NUM_EXPERTS), lambda: (0, 0)),
            pl.BlockSpec((1, 1), lambda: (0, 0)),
            pl.BlockSpec((1, 1), lambda: (0, 0)),
        ],
        out_shape=[
            jax.ShapeDtypeStruct((SEQ, TOP_K), jnp.int32),
            jax.ShapeDtypeStruct((SEQ, TOP_K), jnp.float32),
            jax.ShapeDtypeStruct((1, NUM_EXPERTS), jnp.int32),
            jax.ShapeDtypeStruct((1, NUM_EXPERTS), jnp.int32),
            jax.ShapeDtypeStruct((1, 1), jnp.float32),
            jax.ShapeDtypeStruct((1, 1), jnp.float32),
        ],
    )(x2d, Wr)


@functools.cache
def _sc_scatter_kernel():
    mesh = plsc.VectorSubcoreMesh(core_axis_name="c", subcore_axis_name="s")

    @functools.partial(
        pl.kernel,
        out_type=jax.ShapeDtypeStruct((CAP, EMBED_DIM), jnp.float32),
        mesh=mesh,
        scratch_types=[
            pltpu.VMEM((PW,), jnp.int32),
            pltpu.VMEM((PW, EMBED_DIM), jnp.float32),
            pltpu.SemaphoreType.DMA,
        ],
    )
    def _sc_scatter(x_hbm, destflat_hbm, xg_hbm, idx_v, rows_v, sem):
        wid = lax.axis_index("s") * 2 + lax.axis_index("c")
        base = wid * PW
        tbase = lax.rem(base, SEQ)
        pltpu.sync_copy(destflat_hbm.at[pl.ds(base, PW)], idx_v)
        pltpu.sync_copy(x_hbm.at[pl.ds(tbase, PW)], rows_v)
        pltpu.async_copy(rows_v, xg_hbm.at[idx_v], sem).wait()

    return _sc_scatter


MAXB = SEQ // BLK


def _ffn_body(segblk_ref, nblk_ref, xg_ref, w1_ref, b1_ref, w2_ref, b2_ref,
              yg_ref):
    e = pl.program_id(0)
    j = pl.program_id(1)

    @pl.when(j < nblk_ref[e])
    def _():
        h = jnp.dot(xg_ref[...], w1_ref[0],
                    preferred_element_type=jnp.float32)
        h = _gelu_exact(h + b1_ref[0])
        yg_ref[...] = (jnp.dot(h, w2_ref[0],
                               preferred_element_type=jnp.float32)
                       + b2_ref[0])


@jax.jit
def _ffn_call(segblk, nblk, xg, W1, b1r, W2, b2r):
    def xg_map(e, j, segblk, nblk):
        return (jnp.where(j < nblk[e], segblk[e] + j, 0), 0)

    def yg_map(e, j, segblk, nblk):
        return (jnp.where(j < nblk[e], segblk[e] + j, NB), 0)

    grid_spec = pltpu.PrefetchScalarGridSpec(
        num_scalar_prefetch=2,
        grid=(NUM_EXPERTS, MAXB),
        in_specs=[
            pl.BlockSpec((BLK, EMBED_DIM), xg_map),
            pl.BlockSpec((1, EMBED_DIM, FFN_DIM),
                         lambda e, j, segblk, nblk: (e, 0, 0)),
            pl.BlockSpec((1, 1, FFN_DIM),
                         lambda e, j, segblk, nblk: (e, 0, 0)),
            pl.BlockSpec((1, FFN_DIM, EMBED_DIM),
                         lambda e, j, segblk, nblk: (e, 0, 0)),
            pl.BlockSpec((1, 1, EMBED_DIM),
                         lambda e, j, segblk, nblk: (e, 0, 0)),
        ],
        out_specs=pl.BlockSpec((BLK, EMBED_DIM), yg_map),
    )
    return pl.pallas_call(
        _ffn_body,
        grid_spec=grid_spec,
        out_shape=jax.ShapeDtypeStruct((CAP + BLK, EMBED_DIM), jnp.float32),
        compiler_params=pltpu.CompilerParams(
            dimension_semantics=("arbitrary", "arbitrary")),
    )(segblk, nblk, xg, W1, b1r, W2, b2r)


@functools.cache
def _sc_combine_kernel():
    mesh = plsc.VectorSubcoreMesh(core_axis_name="c", subcore_axis_name="s")

    @functools.partial(
        pl.kernel,
        out_type=jax.ShapeDtypeStruct((SEQ, EMBED_DIM), jnp.float32),
        mesh=mesh,
        scratch_types=[
            pltpu.VMEM((TW,), jnp.int32),
            pltpu.VMEM((TW,), jnp.int32),
            pltpu.VMEM((TW,), jnp.float32),
            pltpu.VMEM((TW,), jnp.float32),
            pltpu.VMEM((TW, EMBED_DIM), jnp.float32),
            pltpu.VMEM((TW, EMBED_DIM), jnp.float32),
            pltpu.SemaphoreType.DMA,
        ],
    )
    def _sc_combine(yg_hbm, d0_hbm, d1_hbm, w0_hbm, w1_hbm, out_hbm,
                    i0_v, i1_v, wa_v, wb_v, r0_v, r1_v, sem):
        wid = lax.axis_index("s") * 2 + lax.axis_index("c")
        base = wid * TW
        pltpu.sync_copy(d0_hbm.at[pl.ds(base, TW)], i0_v)
        pltpu.sync_copy(d1_hbm.at[pl.ds(base, TW)], i1_v)
        pltpu.sync_copy(w0_hbm.at[pl.ds(base, TW)], wa_v)
        pltpu.sync_copy(w1_hbm.at[pl.ds(base, TW)], wb_v)
        pltpu.async_copy(yg_hbm.at[i0_v], r0_v, sem).wait()
        pltpu.async_copy(yg_hbm.at[i1_v], r1_v, sem).wait()

        def gbody(g, carry):
            wa16 = wa_v[pl.ds(g * 16, 16)]
            wb16 = wb_v[pl.ds(g * 16, 16)]

            def cbody(c, carry2):
                sl = pl.ds(c * 16, 16)
                for jj in range(16):
                    j = g * 16 + jj
                    r0_v[j, sl] = (r0_v[j, sl] * wa16[jj]
                                   + r1_v[j, sl] * wb16[jj])
                return carry2

            lax.fori_loop(0, EMBED_DIM // 16, cbody, 0)
            return carry

        lax.fori_loop(0, TW // 16, gbody, 0)
        pltpu.sync_copy(r0_v, out_hbm.at[pl.ds(base, TW)])

    return _sc_combine


def kernel(x, Wr, W1, b1, W2, b2):
    x2d = x.reshape(SEQ, EMBED_DIM)
    dest, wk, segblk, nblk, lb, z = _router_call(x2d, Wr)
    destflat = jnp.concatenate([dest[:, 0], dest[:, 1]], axis=0)
    xg = _sc_scatter_kernel()(x2d, destflat)
    yg = _ffn_call(segblk.reshape(NUM_EXPERTS), nblk.reshape(NUM_EXPERTS),
                   xg, W1, b1.reshape(NUM_EXPERTS, 1, FFN_DIM),
                   W2, b2.reshape(NUM_EXPERTS, 1, EMBED_DIM))
    out = _sc_combine_kernel()(yg, dest[:, 0], dest[:, 1], wk[:, 0], wk[:, 1])
    return out.reshape(x.shape), lb[0, 0], z[0, 0]


# speedup vs baseline: 1.0933x; 1.0933x over previous
import functools
import math

import jax
import jax.numpy as jnp
from jax import lax
from jax.experimental import pallas as pl
from jax.experimental.pallas import tpu as pltpu
from jax.experimental.pallas import tpu_sc as plsc

EMBED_DIM = 768
FFN_DIM = 3072
NUM_EXPERTS = 8
TOP_K = 2
LB_W = 0.01
Z_W = 0.001

SEQ = 2048
BLK = 256
CAP = TOP_K * SEQ + NUM_EXPERTS * BLK
NB = CAP // BLK
NPAIRS = TOP_K * SEQ
NWORKERS = 32
PW = NPAIRS // NWORKERS
TW = SEQ // NWORKERS

_INV_SQRT2 = 1.0 / math.sqrt(2.0)


def _gelu_exact(h):
    return 0.5 * h * (1.0 + jax.lax.erf(h * _INV_SQRT2))


def _cumsum_rows(a):
    n = a.shape[0]
    s = 1
    while s < n:
        shifted = jnp.concatenate(
            [jnp.zeros((s,) + a.shape[1:], a.dtype), a[:n - s]], axis=0)
        a = a + shifted
        s *= 2
    return a


def _router_body(x_ref, wr_ref, dest_ref, wk_ref, be_ref, nv_ref,
                 lb_ref, z_ref):
    xv = x_ref[...]
    logits = jnp.dot(xv, wr_ref[...], preferred_element_type=jnp.float32)
    S, E = logits.shape
    m = jnp.max(logits, axis=1, keepdims=True)
    ex = jnp.exp(logits - m)
    sm = jnp.sum(ex, axis=1, keepdims=True)
    probs = ex / sm
    iota8 = jax.lax.broadcasted_iota(jnp.int32, (S, E), 1)
    m1 = jnp.max(probs, axis=1, keepdims=True)
    i1 = jnp.min(jnp.where(probs == m1, iota8, E), axis=1, keepdims=True)
    probs_m = jnp.where(iota8 == i1, -1.0, probs)
    m2 = jnp.max(probs_m, axis=1, keepdims=True)
    i2 = jnp.min(jnp.where(probs_m == m2, iota8, E), axis=1, keepdims=True)
    denom = m1 + m2 + 1e-8
    wk_ref[...] = jnp.concatenate([m1 / denom, m2 / denom], axis=1)

    usage = jnp.mean(probs, axis=0, keepdims=True)
    mean_u = jnp.mean(usage)
    var_u = jnp.mean(jnp.square(usage - mean_u))
    lb_ref[...] = jnp.reshape(
        var_u / (mean_u * mean_u + 1e-8) * (NUM_EXPERTS * LB_W), (1, 1))
    lse = m[:, 0] + jnp.log(jnp.sum(ex, axis=1))
    z_ref[...] = jnp.reshape(jnp.mean(jnp.square(lse)) * Z_W, (1, 1))

    oh1 = (iota8 == i1).astype(jnp.float32)
    oh2 = (iota8 == i2).astype(jnp.float32)
    c1 = _cumsum_rows(oh1)
    c2 = _cumsum_rows(oh2)
    cnt1 = c1[S - 1:S, :]
    cnt2 = c2[S - 1:S, :]
    rank1 = jnp.sum(oh1 * c1, axis=1, keepdims=True) - 1.0
    rank2 = jnp.sum(oh2 * c2, axis=1, keepdims=True) - 1.0
    cnt = (cnt1 + cnt2).astype(jnp.int32)
    capi = ((cnt + (BLK - 1)) // BLK) * BLK
    csum = capi
    for s in (1, 2, 4):
        csum = csum + jnp.concatenate(
            [jnp.zeros((1, s), jnp.int32), csum[:, :NUM_EXPERTS - s]], axis=1)
    seg_start = csum - capi
    segf = seg_start.astype(jnp.float32)
    base1 = jnp.sum(oh1 * segf, axis=1, keepdims=True)
    base2 = jnp.sum(oh2 * (segf + cnt1), axis=1, keepdims=True)
    dest1 = (base1 + rank1).astype(jnp.int32)
    dest2 = (base2 + rank2).astype(jnp.int32)
    dest_ref[...] = jnp.concatenate([dest1, dest2], axis=1)

    seg_end = seg_start + capi
    bs = jax.lax.broadcasted_iota(jnp.int32, (1, NB), 1) * BLK
    be = jnp.zeros((1, NB), jnp.int32)
    for e in range(NUM_EXPERTS):
        be = be + (bs >= seg_end[0:1, e:e + 1]).astype(jnp.int32)
    be_ref[...] = jnp.minimum(be, NUM_EXPERTS - 1)
    nv_ref[...] = jnp.reshape(jnp.sum(capi) // BLK, (1, 1))


@jax.jit
def _router_call(x2d, Wr):
    return pl.pallas_call(
        _router_body,
        in_specs=[
            pl.BlockSpec((SEQ, EMBED_DIM), lambda: (0, 0)),
            pl.BlockSpec((EMBED_DIM, NUM_EXPERTS), lambda: (0, 0)),
        ],
        out_specs=[
            pl.BlockSpec((SEQ, TOP_K), lambda: (0, 0)),
            pl.BlockSpec((SEQ, TOP_K), lambda: (0, 0)),
            pl.BlockSpec((1, NB), lambda: (0, 0)),
            pl.BlockSpec((1, 1), lambda: (0, 0)),
            pl.BlockSpec((1, 1), lambda: (0, 0)),
            pl.BlockSpec((1, 1), lambda: (0, 0)),
        ],
        out_shape=[
            jax.ShapeDtypeStruct((SEQ, TOP_K), jnp.int32),
            jax.ShapeDtypeStruct((SEQ, TOP_K), jnp.float32),
            jax.ShapeDtypeStruct((1, NB), jnp.int32),
            jax.ShapeDtypeStruct((1, 1), jnp.int32),
            jax.ShapeDtypeStruct((1, 1), jnp.float32),
            jax.ShapeDtypeStruct((1, 1), jnp.float32),
        ],
    )(x2d, Wr)


@functools.cache
def _sc_scatter_kernel():
    mesh = plsc.VectorSubcoreMesh(core_axis_name="c", subcore_axis_name="s")

    @functools.partial(
        pl.kernel,
        out_type=jax.ShapeDtypeStruct((CAP, EMBED_DIM), jnp.float32),
        mesh=mesh,
        scratch_types=[
            pltpu.VMEM((PW,), jnp.int32),
            pltpu.VMEM((PW, EMBED_DIM), jnp.float32),
            pltpu.SemaphoreType.DMA,
        ],
    )
    def _sc_scatter(x_hbm, destflat_hbm, xg_hbm, idx_v, rows_v, sem):
        wid = lax.axis_index("s") * 2 + lax.axis_index("c")
        base = wid * PW
        tbase = lax.rem(base, SEQ)
        pltpu.sync_copy(destflat_hbm.at[pl.ds(base, PW)], idx_v)
        pltpu.sync_copy(x_hbm.at[pl.ds(tbase, PW)], rows_v)
        pltpu.async_copy(rows_v, xg_hbm.at[idx_v], sem).wait()

    return _sc_scatter


FH = FFN_DIM // 2


def _ffn_body(be_ref, nv_ref, xg_ref, w1a_ref, w1b_ref, b1a_ref, b1b_ref,
              w2a_ref, w2b_ref, b2_ref, yg_ref):
    b = pl.program_id(0)

    @pl.when(b < nv_ref[0])
    def _():
        xv = xg_ref[...]
        h1 = _gelu_exact(
            jnp.dot(xv, w1a_ref[0], preferred_element_type=jnp.float32)
            + b1a_ref[0])
        h2 = _gelu_exact(
            jnp.dot(xv, w1b_ref[0], preferred_element_type=jnp.float32)
            + b1b_ref[0])
        yg_ref[...] = (
            jnp.dot(h1, w2a_ref[0], preferred_element_type=jnp.float32)
            + jnp.dot(h2, w2b_ref[0], preferred_element_type=jnp.float32)
            + b2_ref[0])


@jax.jit
def _ffn_call(be, nv, xg, W1, b1r, W2, b2r):
    grid_spec = pltpu.PrefetchScalarGridSpec(
        num_scalar_prefetch=2,
        grid=(NB,),
        in_specs=[
            pl.BlockSpec((BLK, EMBED_DIM), lambda b, be, nv: (b, 0)),
            pl.BlockSpec((1, EMBED_DIM, FH), lambda b, be, nv: (be[b], 0, 0)),
            pl.BlockSpec((1, EMBED_DIM, FH), lambda b, be, nv: (be[b], 0, 1)),
            pl.BlockSpec((1, 1, FH), lambda b, be, nv: (be[b], 0, 0)),
            pl.BlockSpec((1, 1, FH), lambda b, be, nv: (be[b], 0, 1)),
            pl.BlockSpec((1, FH, EMBED_DIM), lambda b, be, nv: (be[b], 0, 0)),
            pl.BlockSpec((1, FH, EMBED_DIM), lambda b, be, nv: (be[b], 1, 0)),
            pl.BlockSpec((1, 1, EMBED_DIM), lambda b, be, nv: (be[b], 0, 0)),
        ],
        out_specs=pl.BlockSpec((BLK, EMBED_DIM), lambda b, be, nv: (b, 0)),
    )
    return pl.pallas_call(
        _ffn_body,
        grid_spec=grid_spec,
        out_shape=jax.ShapeDtypeStruct((CAP, EMBED_DIM), jnp.float32),
        compiler_params=pltpu.CompilerParams(
            dimension_semantics=("arbitrary",)),
    )(be, nv, xg, W1, W1, b1r, b1r, W2, W2, b2r)


@functools.cache
def _sc_combine_kernel():
    mesh = plsc.VectorSubcoreMesh(core_axis_name="c", subcore_axis_name="s")

    @functools.partial(
        pl.kernel,
        out_type=jax.ShapeDtypeStruct((SEQ, EMBED_DIM), jnp.float32),
        mesh=mesh,
        scratch_types=[
            pltpu.VMEM((TW,), jnp.int32),
            pltpu.VMEM((TW,), jnp.int32),
            pltpu.VMEM((TW,), jnp.float32),
            pltpu.VMEM((TW,), jnp.float32),
            pltpu.VMEM((TW, EMBED_DIM), jnp.float32),
            pltpu.VMEM((TW, EMBED_DIM), jnp.float32),
            pltpu.SemaphoreType.DMA,
        ],
    )
    def _sc_combine(yg_hbm, d0_hbm, d1_hbm, w0_hbm, w1_hbm, out_hbm,
                    i0_v, i1_v, wa_v, wb_v, r0_v, r1_v, sem):
        wid = lax.axis_index("s") * 2 + lax.axis_index("c")
        base = wid * TW
        pltpu.sync_copy(d0_hbm.at[pl.ds(base, TW)], i0_v)
        pltpu.sync_copy(d1_hbm.at[pl.ds(base, TW)], i1_v)
        pltpu.sync_copy(w0_hbm.at[pl.ds(base, TW)], wa_v)
        pltpu.sync_copy(w1_hbm.at[pl.ds(base, TW)], wb_v)
        pltpu.async_copy(yg_hbm.at[i0_v], r0_v, sem).wait()
        pltpu.async_copy(yg_hbm.at[i1_v], r1_v, sem).wait()

        def gbody(g, carry):
            wa16 = wa_v[pl.ds(g * 16, 16)]
            wb16 = wb_v[pl.ds(g * 16, 16)]

            def cbody(c, carry2):
                sl = pl.ds(c * 16, 16)
                for jj in range(16):
                    j = g * 16 + jj
                    r0_v[j, sl] = (r0_v[j, sl] * wa16[jj]
                                   + r1_v[j, sl] * wb16[jj])
                return carry2

            lax.fori_loop(0, EMBED_DIM // 16, cbody, 0)
            return carry

        lax.fori_loop(0, TW // 16, gbody, 0)
        pltpu.sync_copy(r0_v, out_hbm.at[pl.ds(base, TW)])

    return _sc_combine


def kernel(x, Wr, W1, b1, W2, b2):
    x2d = x.reshape(SEQ, EMBED_DIM)
    dest, wk, be, nv, lb, z = _router_call(x2d, Wr)
    destflat = jnp.concatenate([dest[:, 0], dest[:, 1]], axis=0)
    xg = _sc_scatter_kernel()(x2d, destflat)
    yg = _ffn_call(be.reshape(NB), nv.reshape(1), xg,
                   W1, b1.reshape(NUM_EXPERTS, 1, FFN_DIM),
                   W2, b2.reshape(NUM_EXPERTS, 1, EMBED_DIM))
    out = _sc_combine_kernel()(yg, dest[:, 0], dest[:, 1], wk[:, 0], wk[:, 1])
    return out.reshape(x.shape), lb[0, 0], z[0, 0]


# speedup vs baseline: 1.1559x; 1.0573x over previous
import functools
import math

import jax
import jax.numpy as jnp
from jax import lax
from jax.experimental import pallas as pl
from jax.experimental.pallas import tpu as pltpu
from jax.experimental.pallas import tpu_sc as plsc

EMBED_DIM = 768
FFN_DIM = 3072
NUM_EXPERTS = 8
TOP_K = 2
LB_W = 0.01
Z_W = 0.001

SEQ = 2048
BLK = 256
CAP = TOP_K * SEQ + NUM_EXPERTS * BLK
NB = CAP // BLK
NPAIRS = TOP_K * SEQ
NWORKERS = 32
PW = NPAIRS // NWORKERS
TW = SEQ // NWORKERS

_INV_SQRT2 = 1.0 / math.sqrt(2.0)


def _gelu_exact(h):
    return 0.5 * h * (1.0 + jax.lax.erf(h * _INV_SQRT2))


def _cumsum_rows(a):
    n = a.shape[0]
    s = 1
    while s < n:
        shifted = jnp.concatenate(
            [jnp.zeros((s,) + a.shape[1:], a.dtype), a[:n - s]], axis=0)
        a = a + shifted
        s *= 2
    return a


def _router_body(x_ref, wr_ref, dest_ref, wk_ref, be_ref, nv_ref,
                 lb_ref, z_ref):
    xv = x_ref[...]
    logits = jnp.dot(xv, wr_ref[...], preferred_element_type=jnp.float32)
    S, E = logits.shape
    m = jnp.max(logits, axis=1, keepdims=True)
    ex = jnp.exp(logits - m)
    sm = jnp.sum(ex, axis=1, keepdims=True)
    probs = ex / sm
    iota8 = jax.lax.broadcasted_iota(jnp.int32, (S, E), 1)
    m1 = jnp.max(probs, axis=1, keepdims=True)
    i1 = jnp.min(jnp.where(probs == m1, iota8, E), axis=1, keepdims=True)
    probs_m = jnp.where(iota8 == i1, -1.0, probs)
    m2 = jnp.max(probs_m, axis=1, keepdims=True)
    i2 = jnp.min(jnp.where(probs_m == m2, iota8, E), axis=1, keepdims=True)
    denom = m1 + m2 + 1e-8
    wk_ref[...] = jnp.concatenate([m1 / denom, m2 / denom], axis=1)

    usage = jnp.mean(probs, axis=0, keepdims=True)
    mean_u = jnp.mean(usage)
    var_u = jnp.mean(jnp.square(usage - mean_u))
    lb_ref[...] = jnp.reshape(
        var_u / (mean_u * mean_u + 1e-8) * (NUM_EXPERTS * LB_W), (1, 1))
    lse = m[:, 0] + jnp.log(jnp.sum(ex, axis=1))
    z_ref[...] = jnp.reshape(jnp.mean(jnp.square(lse)) * Z_W, (1, 1))

    oh1 = (iota8 == i1).astype(jnp.float32)
    oh2 = (iota8 == i2).astype(jnp.float32)
    c1 = _cumsum_rows(oh1)
    c2 = _cumsum_rows(oh2)
    cnt1 = c1[S - 1:S, :]
    cnt2 = c2[S - 1:S, :]
    rank1 = jnp.sum(oh1 * c1, axis=1, keepdims=True) - 1.0
    rank2 = jnp.sum(oh2 * c2, axis=1, keepdims=True) - 1.0
    cnt = (cnt1 + cnt2).astype(jnp.int32)
    capi = ((cnt + (BLK - 1)) // BLK) * BLK
    csum = capi
    for s in (1, 2, 4):
        csum = csum + jnp.concatenate(
            [jnp.zeros((1, s), jnp.int32), csum[:, :NUM_EXPERTS - s]], axis=1)
    seg_start = csum - capi
    segf = seg_start.astype(jnp.float32)
    base1 = jnp.sum(oh1 * segf, axis=1, keepdims=True)
    base2 = jnp.sum(oh2 * (segf + cnt1), axis=1, keepdims=True)
    dest1 = (base1 + rank1).astype(jnp.int32)
    dest2 = (base2 + rank2).astype(jnp.int32)
    dest_ref[...] = jnp.concatenate([dest1, dest2], axis=1)

    seg_end = seg_start + capi
    bs = jax.lax.broadcasted_iota(jnp.int32, (1, NB), 1) * BLK
    be = jnp.zeros((1, NB), jnp.int32)
    for e in range(NUM_EXPERTS):
        be = be + (bs >= seg_end[0:1, e:e + 1]).astype(jnp.int32)
    be_ref[...] = jnp.minimum(be, NUM_EXPERTS - 1)
    nv_ref[...] = jnp.reshape(jnp.sum(capi) // BLK, (1, 1))


@jax.jit
def _router_call(x2d, Wr):
    return pl.pallas_call(
        _router_body,
        in_specs=[
            pl.BlockSpec((SEQ, EMBED_DIM), lambda: (0, 0)),
            pl.BlockSpec((EMBED_DIM, NUM_EXPERTS), lambda: (0, 0)),
        ],
        out_specs=[
            pl.BlockSpec((SEQ, TOP_K), lambda: (0, 0)),
            pl.BlockSpec((SEQ, TOP_K), lambda: (0, 0)),
            pl.BlockSpec((1, NB), lambda: (0, 0)),
            pl.BlockSpec((1, 1), lambda: (0, 0)),
            pl.BlockSpec((1, 1), lambda: (0, 0)),
            pl.BlockSpec((1, 1), lambda: (0, 0)),
        ],
        out_shape=[
            jax.ShapeDtypeStruct((SEQ, TOP_K), jnp.int32),
            jax.ShapeDtypeStruct((SEQ, TOP_K), jnp.float32),
            jax.ShapeDtypeStruct((1, NB), jnp.int32),
            jax.ShapeDtypeStruct((1, 1), jnp.int32),
            jax.ShapeDtypeStruct((1, 1), jnp.float32),
            jax.ShapeDtypeStruct((1, 1), jnp.float32),
        ],
    )(x2d, Wr)


@functools.cache
def _sc_scatter_kernel():
    mesh = plsc.VectorSubcoreMesh(core_axis_name="c", subcore_axis_name="s")

    @functools.partial(
        pl.kernel,
        out_type=jax.ShapeDtypeStruct((CAP, EMBED_DIM), jnp.float32),
        mesh=mesh,
        scratch_types=[
            pltpu.VMEM((PW,), jnp.int32),
            pltpu.VMEM((PW, EMBED_DIM), jnp.float32),
            pltpu.SemaphoreType.DMA,
        ],
    )
    def _sc_scatter(x_hbm, destflat_hbm, xg_hbm, idx_v, rows_v, sem):
        wid = lax.axis_index("s") * 2 + lax.axis_index("c")
        base = wid * PW
        tbase = lax.rem(base, SEQ)
        pltpu.sync_copy(destflat_hbm.at[pl.ds(base, PW)], idx_v)
        pltpu.sync_copy(x_hbm.at[pl.ds(tbase, PW)], rows_v)
        pltpu.async_copy(rows_v, xg_hbm.at[idx_v], sem).wait()

    return _sc_scatter


def _ffn_body(be_ref, nv_ref, xg_ref, w1_ref, b1_ref, w2_ref, b2_ref,
              yg_ref):
    b = pl.program_id(0)

    @pl.when(b < nv_ref[0])
    def _():
        h = jnp.dot(xg_ref[...], w1_ref[0],
                    preferred_element_type=jnp.float32)
        h = _gelu_exact(h + b1_ref[0])
        yg_ref[...] = (jnp.dot(h, w2_ref[0],
                               preferred_element_type=jnp.float32)
                       + b2_ref[0])


@jax.jit
def _ffn_call(be, nv, xg, W1, b1r, W2, b2r):
    def xg_map(b, be, nv):
        return (jnp.where(b < nv[0], b, 0), 0)

    def yg_map(b, be, nv):
        return (jnp.where(b < nv[0], b, jnp.minimum(nv[0], NB - 1)), 0)

    grid_spec = pltpu.PrefetchScalarGridSpec(
        num_scalar_prefetch=2,
        grid=(NB,),
        in_specs=[
            pl.BlockSpec((BLK, EMBED_DIM), xg_map),
            pl.BlockSpec((1, EMBED_DIM, FFN_DIM),
                         lambda b, be, nv: (be[b], 0, 0)),
            pl.BlockSpec((1, 1, FFN_DIM), lambda b, be, nv: (be[b], 0, 0)),
            pl.BlockSpec((1, FFN_DIM, EMBED_DIM),
                         lambda b, be, nv: (be[b], 0, 0)),
            pl.BlockSpec((1, 1, EMBED_DIM), lambda b, be, nv: (be[b], 0, 0)),
        ],
        out_specs=pl.BlockSpec((BLK, EMBED_DIM), yg_map),
    )
    return pl.pallas_call(
        _ffn_body,
        grid_spec=grid_spec,
        out_shape=jax.ShapeDtypeStruct((CAP, EMBED_DIM), jnp.float32),
        compiler_params=pltpu.CompilerParams(
            dimension_semantics=("arbitrary",)),
    )(be, nv, xg, W1, b1r, W2, b2r)


@functools.cache
def _sc_combine_kernel():
    mesh = plsc.VectorSubcoreMesh(core_axis_name="c", subcore_axis_name="s")

    @functools.partial(
        pl.kernel,
        out_type=jax.ShapeDtypeStruct((SEQ, EMBED_DIM), jnp.float32),
        mesh=mesh,
        scratch_types=[
            pltpu.VMEM((TW,), jnp.int32),
            pltpu.VMEM((TW,), jnp.int32),
            pltpu.VMEM((TW,), jnp.float32),
            pltpu.VMEM((TW,), jnp.float32),
            pltpu.VMEM((TW, EMBED_DIM), jnp.float32),
            pltpu.VMEM((TW, EMBED_DIM), jnp.float32),
            pltpu.SemaphoreType.DMA,
        ],
    )
    def _sc_combine(yg_hbm, d0_hbm, d1_hbm, w0_hbm, w1_hbm, out_hbm,
                    i0_v, i1_v, wa_v, wb_v, r0_v, r1_v, sem):
        wid = lax.axis_index("s") * 2 + lax.axis_index("c")
        base = wid * TW
        pltpu.sync_copy(d0_hbm.at[pl.ds(base, TW)], i0_v)
        pltpu.sync_copy(d1_hbm.at[pl.ds(base, TW)], i1_v)
        pltpu.sync_copy(w0_hbm.at[pl.ds(base, TW)], wa_v)
        pltpu.sync_copy(w1_hbm.at[pl.ds(base, TW)], wb_v)
        pltpu.async_copy(yg_hbm.at[i0_v], r0_v, sem).wait()
        pltpu.async_copy(yg_hbm.at[i1_v], r1_v, sem).wait()

        def gbody(g, carry):
            wa16 = wa_v[pl.ds(g * 16, 16)]
            wb16 = wb_v[pl.ds(g * 16, 16)]

            def cbody(c, carry2):
                sl = pl.ds(c * 16, 16)
                for jj in range(16):
                    j = g * 16 + jj
                    r0_v[j, sl] = (r0_v[j, sl] * wa16[jj]
                                   + r1_v[j, sl] * wb16[jj])
                return carry2

            lax.fori_loop(0, EMBED_DIM // 16, cbody, 0)
            return carry

        lax.fori_loop(0, TW // 16, gbody, 0)
        pltpu.sync_copy(r0_v, out_hbm.at[pl.ds(base, TW)])

    return _sc_combine


def kernel(x, Wr, W1, b1, W2, b2):
    x2d = x.reshape(SEQ, EMBED_DIM)
    dest, wk, be, nv, lb, z = _router_call(x2d, Wr)
    destflat = jnp.concatenate([dest[:, 0], dest[:, 1]], axis=0)
    xg = _sc_scatter_kernel()(x2d, destflat)
    yg = _ffn_call(be.reshape(NB), nv.reshape(1), xg,
                   W1, b1.reshape(NUM_EXPERTS, 1, FFN_DIM),
                   W2, b2.reshape(NUM_EXPERTS, 1, EMBED_DIM))
    out = _sc_combine_kernel()(yg, dest[:, 0], dest[:, 1], wk[:, 0], wk[:, 1])
    return out.reshape(x.shape), lb[0, 0], z[0, 0]


# speedup vs baseline: 1.1696x; 1.0119x over previous
import functools
import math

import jax
import jax.numpy as jnp
from jax import lax
from jax.experimental import pallas as pl
from jax.experimental.pallas import tpu as pltpu
from jax.experimental.pallas import tpu_sc as plsc

EMBED_DIM = 768
FFN_DIM = 3072
NUM_EXPERTS = 8
TOP_K = 2
LB_W = 0.01
Z_W = 0.001

SEQ = 2048
BLK = 256
CAP = TOP_K * SEQ + NUM_EXPERTS * BLK
NB = CAP // BLK
NPAIRS = TOP_K * SEQ
NWORKERS = 32
PW = NPAIRS // NWORKERS
TW = SEQ // NWORKERS

_INV_SQRT2 = 1.0 / math.sqrt(2.0)


def _gelu_exact(h):
    return 0.5 * h * (1.0 + jax.lax.erf(h * _INV_SQRT2))


def _cumsum_rows(a):
    n = a.shape[0]
    s = 1
    while s < n:
        shifted = jnp.concatenate(
            [jnp.zeros((s,) + a.shape[1:], a.dtype), a[:n - s]], axis=0)
        a = a + shifted
        s *= 2
    return a


def _router_body(x_ref, wr_ref, dest_ref, wk_ref, be_ref, nv_ref,
                 lb_ref, z_ref):
    xv = x_ref[...]
    logits = jnp.dot(xv, wr_ref[...], preferred_element_type=jnp.float32)
    S, E = logits.shape
    m = jnp.max(logits, axis=1, keepdims=True)
    ex = jnp.exp(logits - m)
    sm = jnp.sum(ex, axis=1, keepdims=True)
    probs = ex / sm
    iota8 = jax.lax.broadcasted_iota(jnp.int32, (S, E), 1)
    m1 = jnp.max(probs, axis=1, keepdims=True)
    i1 = jnp.min(jnp.where(probs == m1, iota8, E), axis=1, keepdims=True)
    probs_m = jnp.where(iota8 == i1, -1.0, probs)
    m2 = jnp.max(probs_m, axis=1, keepdims=True)
    i2 = jnp.min(jnp.where(probs_m == m2, iota8, E), axis=1, keepdims=True)
    denom = m1 + m2 + 1e-8
    wk_ref[...] = jnp.concatenate(
        [jnp.reshape(m1 / denom, (1, S)), jnp.reshape(m2 / denom, (1, S))],
        axis=0)

    usage = jnp.mean(probs, axis=0, keepdims=True)
    mean_u = jnp.mean(usage)
    var_u = jnp.mean(jnp.square(usage - mean_u))
    lb_ref[...] = jnp.reshape(
        var_u / (mean_u * mean_u + 1e-8) * (NUM_EXPERTS * LB_W), (1, 1))
    lse = m[:, 0] + jnp.log(jnp.sum(ex, axis=1))
    z_ref[...] = jnp.reshape(jnp.mean(jnp.square(lse)) * Z_W, (1, 1))

    oh1 = (iota8 == i1).astype(jnp.float32)
    oh2 = (iota8 == i2).astype(jnp.float32)
    c1 = _cumsum_rows(oh1)
    c2 = _cumsum_rows(oh2)
    cnt1 = c1[S - 1:S, :]
    cnt2 = c2[S - 1:S, :]
    rank1 = jnp.sum(oh1 * c1, axis=1, keepdims=True) - 1.0
    rank2 = jnp.sum(oh2 * c2, axis=1, keepdims=True) - 1.0
    cnt = (cnt1 + cnt2).astype(jnp.int32)
    capi = ((cnt + (BLK - 1)) // BLK) * BLK
    csum = capi
    for s in (1, 2, 4):
        csum = csum + jnp.concatenate(
            [jnp.zeros((1, s), jnp.int32), csum[:, :NUM_EXPERTS - s]], axis=1)
    seg_start = csum - capi
    segf = seg_start.astype(jnp.float32)
    base1 = jnp.sum(oh1 * segf, axis=1, keepdims=True)
    base2 = jnp.sum(oh2 * (segf + cnt1), axis=1, keepdims=True)
    dest1 = (base1 + rank1).astype(jnp.int32)
    dest2 = (base2 + rank2).astype(jnp.int32)
    dest_ref[...] = jnp.concatenate(
        [jnp.reshape(dest1, (1, S)), jnp.reshape(dest2, (1, S))], axis=0)

    seg_end = seg_start + capi
    bs = jax.lax.broadcasted_iota(jnp.int32, (1, NB), 1) * BLK
    be = jnp.zeros((1, NB), jnp.int32)
    for e in range(NUM_EXPERTS):
        be = be + (bs >= seg_end[0:1, e:e + 1]).astype(jnp.int32)
    be_ref[...] = jnp.minimum(be, NUM_EXPERTS - 1)
    nv_ref[...] = jnp.reshape(jnp.sum(capi) // BLK, (1, 1))


@jax.jit
def _router_call(x2d, Wr):
    return pl.pallas_call(
        _router_body,
        in_specs=[
            pl.BlockSpec((SEQ, EMBED_DIM), lambda: (0, 0)),
            pl.BlockSpec((EMBED_DIM, NUM_EXPERTS), lambda: (0, 0)),
        ],
        out_specs=[
            pl.BlockSpec((TOP_K, SEQ), lambda: (0, 0)),
            pl.BlockSpec((TOP_K, SEQ), lambda: (0, 0)),
            pl.BlockSpec((1, NB), lambda: (0, 0)),
            pl.BlockSpec((1, 1), lambda: (0, 0)),
            pl.BlockSpec((1, 1), lambda: (0, 0)),
            pl.BlockSpec((1, 1), lambda: (0, 0)),
        ],
        out_shape=[
            jax.ShapeDtypeStruct((TOP_K, SEQ), jnp.int32),
            jax.ShapeDtypeStruct((TOP_K, SEQ), jnp.float32),
            jax.ShapeDtypeStruct((1, NB), jnp.int32),
            jax.ShapeDtypeStruct((1, 1), jnp.int32),
            jax.ShapeDtypeStruct((1, 1), jnp.float32),
            jax.ShapeDtypeStruct((1, 1), jnp.float32),
        ],
    )(x2d, Wr)


@functools.cache
def _sc_scatter_kernel():
    mesh = plsc.VectorSubcoreMesh(core_axis_name="c", subcore_axis_name="s")

    @functools.partial(
        pl.kernel,
        out_type=jax.ShapeDtypeStruct((CAP, EMBED_DIM), jnp.float32),
        mesh=mesh,
        scratch_types=[
            pltpu.VMEM((PW,), jnp.int32),
            pltpu.VMEM((PW, EMBED_DIM), jnp.float32),
            pltpu.SemaphoreType.DMA,
        ],
    )
    def _sc_scatter(x_hbm, destflat_hbm, xg_hbm, idx_v, rows_v, sem):
        wid = lax.axis_index("s") * 2 + lax.axis_index("c")
        base = wid * PW
        tbase = lax.rem(base, SEQ)
        pltpu.sync_copy(destflat_hbm.at[pl.ds(base, PW)], idx_v)
        pltpu.sync_copy(x_hbm.at[pl.ds(tbase, PW)], rows_v)
        pltpu.async_copy(rows_v, xg_hbm.at[idx_v], sem).wait()

    return _sc_scatter


def _ffn_body(be_ref, nv_ref, xg_ref, w1_ref, b1_ref, w2_ref, b2_ref,
              yg_ref):
    b = pl.program_id(0)

    @pl.when(b < nv_ref[0])
    def _():
        h = jnp.dot(xg_ref[...], w1_ref[0],
                    preferred_element_type=jnp.float32)
        h = _gelu_exact(h + b1_ref[0])
        yg_ref[...] = (jnp.dot(h, w2_ref[0],
                               preferred_element_type=jnp.float32)
                       + b2_ref[0])


@jax.jit
def _ffn_call(be, nv, xg, W1, b1r, W2, b2r):
    def xg_map(b, be, nv):
        return (jnp.where(b < nv[0], b, 0), 0)

    def yg_map(b, be, nv):
        return (jnp.where(b < nv[0], b, jnp.minimum(nv[0], NB - 1)), 0)

    grid_spec = pltpu.PrefetchScalarGridSpec(
        num_scalar_prefetch=2,
        grid=(NB,),
        in_specs=[
            pl.BlockSpec((BLK, EMBED_DIM), xg_map),
            pl.BlockSpec((1, EMBED_DIM, FFN_DIM),
                         lambda b, be, nv: (be[b], 0, 0)),
            pl.BlockSpec((1, 1, FFN_DIM), lambda b, be, nv: (be[b], 0, 0)),
            pl.BlockSpec((1, FFN_DIM, EMBED_DIM),
                         lambda b, be, nv: (be[b], 0, 0)),
            pl.BlockSpec((1, 1, EMBED_DIM), lambda b, be, nv: (be[b], 0, 0)),
        ],
        out_specs=pl.BlockSpec((BLK, EMBED_DIM), yg_map),
    )
    return pl.pallas_call(
        _ffn_body,
        grid_spec=grid_spec,
        out_shape=jax.ShapeDtypeStruct((CAP, EMBED_DIM), jnp.float32),
        compiler_params=pltpu.CompilerParams(
            dimension_semantics=("arbitrary",)),
    )(be, nv, xg, W1, b1r, W2, b2r)


@functools.cache
def _sc_combine_kernel():
    mesh = plsc.VectorSubcoreMesh(core_axis_name="c", subcore_axis_name="s")

    @functools.partial(
        pl.kernel,
        out_type=jax.ShapeDtypeStruct((SEQ, EMBED_DIM), jnp.float32),
        mesh=mesh,
        scratch_types=[
            pltpu.VMEM((TW,), jnp.int32),
            pltpu.VMEM((TW,), jnp.int32),
            pltpu.VMEM((TW,), jnp.float32),
            pltpu.VMEM((TW,), jnp.float32),
            pltpu.VMEM((TW, EMBED_DIM), jnp.float32),
            pltpu.VMEM((TW, EMBED_DIM), jnp.float32),
            pltpu.SemaphoreType.DMA,
        ],
    )
    def _sc_combine(yg_hbm, destflat_hbm, wflat_hbm, out_hbm,
                    i0_v, i1_v, wa_v, wb_v, r0_v, r1_v, sem):
        wid = lax.axis_index("s") * 2 + lax.axis_index("c")
        base = wid * TW
        pltpu.sync_copy(destflat_hbm.at[pl.ds(base, TW)], i0_v)
        pltpu.sync_copy(destflat_hbm.at[pl.ds(SEQ + base, TW)], i1_v)
        pltpu.sync_copy(wflat_hbm.at[pl.ds(base, TW)], wa_v)
        pltpu.sync_copy(wflat_hbm.at[pl.ds(SEQ + base, TW)], wb_v)
        pltpu.async_copy(yg_hbm.at[i0_v], r0_v, sem).wait()
        pltpu.async_copy(yg_hbm.at[i1_v], r1_v, sem).wait()

        def gbody(g, carry):
            wa16 = wa_v[pl.ds(g * 16, 16)]
            wb16 = wb_v[pl.ds(g * 16, 16)]

            def cbody(c, carry2):
                sl = pl.ds(c * 16, 16)
                for jj in range(16):
                    j = g * 16 + jj
                    r0_v[j, sl] = (r0_v[j, sl] * wa16[jj]
                                   + r1_v[j, sl] * wb16[jj])
                return carry2

            lax.fori_loop(0, EMBED_DIM // 16, cbody, 0)
            return carry

        lax.fori_loop(0, TW // 16, gbody, 0)
        pltpu.sync_copy(r0_v, out_hbm.at[pl.ds(base, TW)])

    return _sc_combine


def kernel(x, Wr, W1, b1, W2, b2):
    x2d = x.reshape(SEQ, EMBED_DIM)
    dest, wk, be, nv, lb, z = _router_call(x2d, Wr)
    destflat = dest.reshape(NPAIRS)
    wflat = wk.reshape(NPAIRS)
    xg = _sc_scatter_kernel()(x2d, destflat)
    yg = _ffn_call(be.reshape(NB), nv.reshape(1), xg,
                   W1, b1.reshape(NUM_EXPERTS, 1, FFN_DIM),
                   W2, b2.reshape(NUM_EXPERTS, 1, EMBED_DIM))
    out = _sc_combine_kernel()(yg, destflat, wflat)
    return out.reshape(x.shape), lb[0, 0], z[0, 0]


# speedup vs baseline: 1.1841x; 1.0124x over previous
import functools
import math

import jax
import jax.numpy as jnp
from jax import lax
from jax.experimental import pallas as pl
from jax.experimental.pallas import tpu as pltpu
from jax.experimental.pallas import tpu_sc as plsc

EMBED_DIM = 768
FFN_DIM = 3072
NUM_EXPERTS = 8
TOP_K = 2
LB_W = 0.01
Z_W = 0.001

SEQ = 2048
BLK = 256
CAP = TOP_K * SEQ + NUM_EXPERTS * BLK
NB = CAP // BLK
NPAIRS = TOP_K * SEQ
NWORKERS = 32
PW = NPAIRS // NWORKERS
TW = SEQ // NWORKERS

_INV_SQRT2 = 1.0 / math.sqrt(2.0)


def _gelu_exact(h):
    return 0.5 * h * (1.0 + jax.lax.erf(h * _INV_SQRT2))


def _cumsum_rows(a):
    n = a.shape[0]
    s = 1
    while s < n:
        shifted = jnp.concatenate(
            [jnp.zeros((s,) + a.shape[1:], a.dtype), a[:n - s]], axis=0)
        a = a + shifted
        s *= 2
    return a


def _router_body(x_ref, wr_ref, dest_ref, wk_ref, be_ref, nv_ref,
                 lb_ref, z_ref):
    xv = x_ref[...]
    logits = jnp.dot(xv, wr_ref[...], preferred_element_type=jnp.float32)
    S, E = logits.shape
    m = jnp.max(logits, axis=1, keepdims=True)
    ex = jnp.exp(logits - m)
    sm = jnp.sum(ex, axis=1, keepdims=True)
    probs = ex / sm
    iota8 = jax.lax.broadcasted_iota(jnp.int32, (S, E), 1)
    m1 = jnp.max(probs, axis=1, keepdims=True)
    i1 = jnp.min(jnp.where(probs == m1, iota8, E), axis=1, keepdims=True)
    probs_m = jnp.where(iota8 == i1, -1.0, probs)
    m2 = jnp.max(probs_m, axis=1, keepdims=True)
    i2 = jnp.min(jnp.where(probs_m == m2, iota8, E), axis=1, keepdims=True)
    denom = m1 + m2 + 1e-8
    wk_ref[...] = jnp.concatenate(
        [jnp.reshape(m1 / denom, (1, S)), jnp.reshape(m2 / denom, (1, S))],
        axis=0)

    usage = jnp.mean(probs, axis=0, keepdims=True)
    mean_u = jnp.mean(usage)
    var_u = jnp.mean(jnp.square(usage - mean_u))
    lb_ref[...] = jnp.reshape(
        var_u / (mean_u * mean_u + 1e-8) * (NUM_EXPERTS * LB_W), (1, 1))
    lse = m[:, 0] + jnp.log(jnp.sum(ex, axis=1))
    z_ref[...] = jnp.reshape(jnp.mean(jnp.square(lse)) * Z_W, (1, 1))

    oh1 = (iota8 == i1).astype(jnp.float32)
    oh2 = (iota8 == i2).astype(jnp.float32)
    c1 = _cumsum_rows(oh1)
    c2 = _cumsum_rows(oh2)
    cnt1 = c1[S - 1:S, :]
    cnt2 = c2[S - 1:S, :]
    rank1 = jnp.sum(oh1 * c1, axis=1, keepdims=True) - 1.0
    rank2 = jnp.sum(oh2 * c2, axis=1, keepdims=True) - 1.0
    cnt = (cnt1 + cnt2).astype(jnp.int32)
    capi = ((cnt + (BLK - 1)) // BLK) * BLK
    csum = capi
    for s in (1, 2, 4):
        csum = csum + jnp.concatenate(
            [jnp.zeros((1, s), jnp.int32), csum[:, :NUM_EXPERTS - s]], axis=1)
    seg_start = csum - capi
    segf = seg_start.astype(jnp.float32)
    base1 = jnp.sum(oh1 * segf, axis=1, keepdims=True)
    base2 = jnp.sum(oh2 * (segf + cnt1), axis=1, keepdims=True)
    dest1 = (base1 + rank1).astype(jnp.int32)
    dest2 = (base2 + rank2).astype(jnp.int32)
    dest_ref[...] = jnp.concatenate(
        [jnp.reshape(dest1, (1, S)), jnp.reshape(dest2, (1, S))], axis=0)

    seg_end = seg_start + capi
    bs = jax.lax.broadcasted_iota(jnp.int32, (1, NB), 1) * BLK
    be = jnp.zeros((1, NB), jnp.int32)
    for e in range(NUM_EXPERTS):
        be = be + (bs >= seg_end[0:1, e:e + 1]).astype(jnp.int32)
    be_ref[...] = jnp.minimum(be, NUM_EXPERTS - 1)
    nv_ref[...] = jnp.reshape(jnp.sum(capi) // BLK, (1, 1))


@jax.jit
def _router_call(x2d, Wr):
    return pl.pallas_call(
        _router_body,
        in_specs=[
            pl.BlockSpec((SEQ, EMBED_DIM), lambda: (0, 0)),
            pl.BlockSpec((EMBED_DIM, NUM_EXPERTS), lambda: (0, 0)),
        ],
        out_specs=[
            pl.BlockSpec((TOP_K, SEQ), lambda: (0, 0)),
            pl.BlockSpec((TOP_K, SEQ), lambda: (0, 0)),
            pl.BlockSpec((1, NB), lambda: (0, 0)),
            pl.BlockSpec((1, 1), lambda: (0, 0)),
            pl.BlockSpec((1, 1), lambda: (0, 0)),
            pl.BlockSpec((1, 1), lambda: (0, 0)),
        ],
        out_shape=[
            jax.ShapeDtypeStruct((TOP_K, SEQ), jnp.int32),
            jax.ShapeDtypeStruct((TOP_K, SEQ), jnp.float32),
            jax.ShapeDtypeStruct((1, NB), jnp.int32),
            jax.ShapeDtypeStruct((1, 1), jnp.int32),
            jax.ShapeDtypeStruct((1, 1), jnp.float32),
            jax.ShapeDtypeStruct((1, 1), jnp.float32),
        ],
    )(x2d, Wr)


@functools.cache
def _sc_scatter_kernel():
    mesh = plsc.VectorSubcoreMesh(core_axis_name="c", subcore_axis_name="s")

    @functools.partial(
        pl.kernel,
        out_type=jax.ShapeDtypeStruct((CAP, EMBED_DIM), jnp.float32),
        mesh=mesh,
        scratch_types=[
            pltpu.VMEM((PW,), jnp.int32),
            pltpu.VMEM((PW, EMBED_DIM), jnp.float32),
            pltpu.SemaphoreType.DMA,
        ],
    )
    def _sc_scatter(x_hbm, destflat_hbm, xg_hbm, idx_v, rows_v, sem):
        wid = lax.axis_index("s") * 2 + lax.axis_index("c")
        base = wid * PW
        tbase = lax.rem(base, SEQ)
        rows_cp = pltpu.async_copy(x_hbm.at[pl.ds(tbase, PW)], rows_v, sem)
        pltpu.sync_copy(destflat_hbm.at[pl.ds(base, PW)], idx_v)
        rows_cp.wait()
        pltpu.async_copy(rows_v, xg_hbm.at[idx_v], sem).wait()

    return _sc_scatter


def _ffn_body(be_ref, nv_ref, xg_ref, w1_ref, b1_ref, w2_ref, b2_ref,
              yg_ref):
    b = pl.program_id(0)

    @pl.when(b < nv_ref[0])
    def _():
        h = jnp.dot(xg_ref[...], w1_ref[0],
                    preferred_element_type=jnp.float32)
        h = _gelu_exact(h + b1_ref[0])
        yg_ref[...] = (jnp.dot(h, w2_ref[0],
                               preferred_element_type=jnp.float32)
                       + b2_ref[0])


@jax.jit
def _ffn_call(be, nv, xg, W1, b1r, W2, b2r):
    def xg_map(b, be, nv):
        return (jnp.where(b < nv[0], b, 0), 0)

    def yg_map(b, be, nv):
        return (jnp.where(b < nv[0], b, jnp.minimum(nv[0], NB - 1)), 0)

    grid_spec = pltpu.PrefetchScalarGridSpec(
        num_scalar_prefetch=2,
        grid=(NB,),
        in_specs=[
            pl.BlockSpec((BLK, EMBED_DIM), xg_map),
            pl.BlockSpec((1, EMBED_DIM, FFN_DIM),
                         lambda b, be, nv: (be[b], 0, 0)),
            pl.BlockSpec((1, 1, FFN_DIM), lambda b, be, nv: (be[b], 0, 0)),
            pl.BlockSpec((1, FFN_DIM, EMBED_DIM),
                         lambda b, be, nv: (be[b], 0, 0)),
            pl.BlockSpec((1, 1, EMBED_DIM), lambda b, be, nv: (be[b], 0, 0)),
        ],
        out_specs=pl.BlockSpec((BLK, EMBED_DIM), yg_map),
    )
    return pl.pallas_call(
        _ffn_body,
        grid_spec=grid_spec,
        out_shape=jax.ShapeDtypeStruct((CAP, EMBED_DIM), jnp.float32),
        compiler_params=pltpu.CompilerParams(
            dimension_semantics=("arbitrary",)),
    )(be, nv, xg, W1, b1r, W2, b2r)


@functools.cache
def _sc_combine_kernel():
    mesh = plsc.VectorSubcoreMesh(core_axis_name="c", subcore_axis_name="s")

    @functools.partial(
        pl.kernel,
        out_type=jax.ShapeDtypeStruct((SEQ, EMBED_DIM), jnp.float32),
        mesh=mesh,
        scratch_types=[
            pltpu.VMEM((TW,), jnp.int32),
            pltpu.VMEM((TW,), jnp.int32),
            pltpu.VMEM((TW,), jnp.float32),
            pltpu.VMEM((TW,), jnp.float32),
            pltpu.VMEM((TW, EMBED_DIM), jnp.float32),
            pltpu.VMEM((TW, EMBED_DIM), jnp.float32),
            pltpu.SemaphoreType.DMA,
        ],
    )
    def _sc_combine(yg_hbm, destflat_hbm, wflat_hbm, out_hbm,
                    i0_v, i1_v, wa_v, wb_v, r0_v, r1_v, sem):
        wid = lax.axis_index("s") * 2 + lax.axis_index("c")
        base = wid * TW
        pltpu.sync_copy(destflat_hbm.at[pl.ds(base, TW)], i0_v)
        pltpu.sync_copy(destflat_hbm.at[pl.ds(SEQ + base, TW)], i1_v)
        cp0 = pltpu.async_copy(yg_hbm.at[i0_v], r0_v, sem)
        cp1 = pltpu.async_copy(yg_hbm.at[i1_v], r1_v, sem)
        pltpu.sync_copy(wflat_hbm.at[pl.ds(base, TW)], wa_v)
        pltpu.sync_copy(wflat_hbm.at[pl.ds(SEQ + base, TW)], wb_v)
        cp0.wait()
        cp1.wait()

        def gbody(g, carry):
            wa16 = wa_v[pl.ds(g * 16, 16)]
            wb16 = wb_v[pl.ds(g * 16, 16)]

            def cbody(c, carry2):
                sl = pl.ds(c * 16, 16)
                for jj in range(16):
                    j = g * 16 + jj
                    r0_v[j, sl] = (r0_v[j, sl] * wa16[jj]
                                   + r1_v[j, sl] * wb16[jj])
                return carry2

            lax.fori_loop(0, EMBED_DIM // 16, cbody, 0)
            return carry

        lax.fori_loop(0, TW // 16, gbody, 0)
        pltpu.sync_copy(r0_v, out_hbm.at[pl.ds(base, TW)])

    return _sc_combine


def kernel(x, Wr, W1, b1, W2, b2):
    x2d = x.reshape(SEQ, EMBED_DIM)
    dest, wk, be, nv, lb, z = _router_call(x2d, Wr)
    destflat = dest.reshape(NPAIRS)
    wflat = wk.reshape(NPAIRS)
    xg = _sc_scatter_kernel()(x2d, destflat)
    yg = _ffn_call(be.reshape(NB), nv.reshape(1), xg,
                   W1, b1.reshape(NUM_EXPERTS, 1, FFN_DIM),
                   W2, b2.reshape(NUM_EXPERTS, 1, EMBED_DIM))
    out = _sc_combine_kernel()(yg, destflat, wflat)
    return out.reshape(x.shape), lb[0, 0], z[0, 0]
